# baseline (speedup 1.0000x reference)
import functools

import jax
import jax.numpy as jnp
from jax import lax
from jax.experimental import pallas as pl
from jax.experimental.pallas import tpu as pltpu
from jax.experimental.pallas import tpu_sc as plsc

N = 10000
E = 320000
D_IN = 128
H = 64
C = 128

NC = 2
NS = 16
K = 128
GS = 4
KG = K // GS
IB = 16
CH0 = 112
CH1 = 48
CHT = CH0 + CH1
NB0 = CH0 // IB
NB1 = CH1 // IB
E_PAD = K * CHT * NS
N_PAD = 10240
RPT = N_PAD // NS

_mesh = plsc.VectorSubcoreMesh(core_axis_name="c", subcore_axis_name="s")


def _make_segsum(CW, tc_tiling, ch0):
    params = pltpu.CompilerParams(use_tc_tiling_on_sc=tc_tiling)
    nb0, nb1 = ch0 // IB, (CHT - ch0) // IB

    @functools.partial(
        pl.kernel,
        out_type=jax.ShapeDtypeStruct((NC, N_PAD, CW), jnp.float32),
        mesh=_mesh,
        compiler_params=params,
        scratch_types=[
            pltpu.VMEM((IB, K), jnp.int32),
            pltpu.VMEM((IB, K), jnp.int32),
            pltpu.VMEM((K, CW), jnp.float32),
            pltpu.VMEM((K, CW), jnp.float32),
            pltpu.VMEM_SHARED((N_PAD, CW), jnp.float32),
            pltpu.SemaphoreType.DMA,
            pltpu.SemaphoreType.DMA,
            pltpu.SemaphoreType.DMA,
            pltpu.SemaphoreType.DMA,
        ],
    )
    def segsum(h_hbm, srcp_hbm, dstp_hbm, zeros_hbm, out_hbm,
               src_v, dst_v, rows0, rows1, acc, gsem0, gsem1, ssem0, ssem1):
        cid = lax.axis_index("c")
        sid = lax.axis_index("s")
        base = cid * ch0
        nbs = jnp.where(cid == 0, nb0, nb1)
        r0 = sid * RPT
        pltpu.sync_copy(zeros_hbm, acc.at[pl.ds(r0, RPT)])
        plsc.subcore_barrier()

        bufs = ((rows0, gsem0, ssem0), (rows1, gsem1, ssem1))

        def gather(src_v, j, rows, gsem):
            for g in range(GS):
                pltpu.async_copy(h_hbm.at[src_v.at[j, pl.ds(g * KG, KG)]],
                                 rows.at[pl.ds(g * KG, KG)], gsem)

        def gather_wait(src_v, j, rows, gsem):
            for g in range(GS):
                pltpu.make_async_copy(
                    h_hbm.at[src_v.at[j, pl.ds(g * KG, KG)]],
                    rows.at[pl.ds(g * KG, KG)], gsem).wait()

        def block(nb, carry):
            c0 = base + nb * IB
            pltpu.sync_copy(srcp_hbm.at[sid, pl.ds(c0, IB)], src_v)
            pltpu.sync_copy(dstp_hbm.at[sid, pl.ds(c0, IB)], dst_v)
            gather(src_v, 0, rows0, gsem0)
            for j in range(IB):
                rows, gsem, ssem = bufs[j % 2]
                prows, pgsem, pssem = bufs[(j + 1) % 2]
                gather_wait(src_v, j, rows, gsem)
                pltpu.async_copy(rows, acc.at[dst_v.at[j]], ssem, add=True)
                if j >= 1:
                    pltpu.make_async_copy(
                        prows, acc.at[dst_v.at[j - 1]], pssem).wait()
                if j + 1 < IB:
                    gather(src_v, j + 1, prows, pgsem)
            rows, gsem, ssem = bufs[(IB - 1) % 2]
            pltpu.make_async_copy(rows, acc.at[dst_v.at[IB - 1]], ssem).wait()
            return carry

        lax.fori_loop(0, nbs, block, 0)
        plsc.subcore_barrier()
        pltpu.sync_copy(acc.at[pl.ds(r0, RPT)],
                        out_hbm.at[cid, pl.ds(r0, RPT)])

    return segsum


_segsum128 = _make_segsum(128, True, 112)
_segsum64 = _make_segsum(64, False, 128)

CD = 16


@functools.partial(
    pl.kernel,
    out_type=jax.ShapeDtypeStruct((NC, N_PAD, CD), jnp.float32),
    mesh=_mesh,
    compiler_params=pltpu.CompilerParams(use_tc_tiling_on_sc=False),
    scratch_types=[
        pltpu.VMEM((IB, K), jnp.int32),
        pltpu.VMEM((K, CD), jnp.float32),
        pltpu.VMEM_SHARED((N_PAD, CD), jnp.float32),
    ],
)
def _deg(dstp_hbm, ones_hbm, zeros_hbm, out_hbm, dst_v, ones_v, acc):
    cid = lax.axis_index("c")
    sid = lax.axis_index("s")
    base = cid * CH0
    nbs = jnp.where(cid == 0, NB0, NB1)
    pltpu.sync_copy(ones_hbm, ones_v)
    r0 = sid * RPT
    pltpu.sync_copy(zeros_hbm, acc.at[pl.ds(r0, RPT)])
    plsc.subcore_barrier()

    def block(nb, carry):
        pltpu.sync_copy(dstp_hbm.at[sid, pl.ds(base + nb * IB, IB)], dst_v)
        for j in range(IB):
            pltpu.sync_copy(ones_v, acc.at[dst_v.at[j]], add=True)
        return carry

    lax.fori_loop(0, nbs, block, 0)
    plsc.subcore_barrier()
    pltpu.sync_copy(acc.at[pl.ds(r0, RPT)], out_hbm.at[cid, pl.ds(r0, RPT)])


BM = 2048


def _finish_deg_body(deg_ref, o_ref):
    d = deg_ref[...]
    o_ref[...] = lax.rsqrt(d[0, :, 0:1] + d[1, :, 0:1] + 1.0)


_finish_deg = pl.pallas_call(
    _finish_deg_body,
    grid=(N_PAD // BM,),
    in_specs=[pl.BlockSpec((NC, BM, CD), lambda i: (0, i, 0))],
    out_specs=pl.BlockSpec((BM, 1), lambda i: (i, 0)),
    out_shape=jax.ShapeDtypeStruct((N_PAD, 1), jnp.float32),
)


def _mm_scale_body(p_ref, w_ref, dv_ref, o_ref):
    o_ref[...] = dv_ref[...] * jnp.dot(
        p_ref[...], w_ref[...], preferred_element_type=jnp.float32)


def _make_mm_scale(cin, cout):
    return pl.pallas_call(
        _mm_scale_body,
        grid=(N_PAD // BM,),
        in_specs=[
            pl.BlockSpec((BM, cin), lambda i: (i, 0)),
            pl.BlockSpec((cin, cout), lambda i: (0, 0)),
            pl.BlockSpec((BM, 1), lambda i: (i, 0)),
        ],
        out_specs=pl.BlockSpec((BM, cout), lambda i: (i, 0)),
        out_shape=jax.ShapeDtypeStruct((N_PAD, cout), jnp.float32),
    )


_mm_scale_128_64 = _make_mm_scale(128, 64)
_mm_scale_64_64 = _make_mm_scale(64, 64)
_mm_scale_64_128 = _make_mm_scale(64, 128)


def _combine_body(seg_ref, u_ref, dv_ref, b_ref, o_ref, *, crelu):
    t = dv_ref[...] * (seg_ref[0] + seg_ref[1] + u_ref[...]) + b_ref[...]
    if crelu == t.shape[1]:
        t = jnp.maximum(t, 0.0)
    elif crelu > 0:
        cols = lax.broadcasted_iota(jnp.int32, t.shape, 1)
        t = jnp.where(cols < crelu, jnp.maximum(t, 0.0), t)
    o_ref[...] = t


def _make_combine(cw, crelu):
    return pl.pallas_call(
        functools.partial(_combine_body, crelu=crelu),
        grid=(N_PAD // BM,),
        in_specs=[
            pl.BlockSpec((NC, BM, cw), lambda i: (0, i, 0)),
            pl.BlockSpec((BM, cw), lambda i: (i, 0)),
            pl.BlockSpec((BM, 1), lambda i: (i, 0)),
            pl.BlockSpec((1, cw), lambda i: (0, 0)),
        ],
        out_specs=pl.BlockSpec((BM, cw), lambda i: (i, 0)),
        out_shape=jax.ShapeDtypeStruct((N_PAD, cw), jnp.float32),
    )


_combine_64_relu = _make_combine(64, 64)
_combine_64 = _make_combine(64, 0)
_combine_128_relu64 = _make_combine(128, 64)
_combine_128 = _make_combine(128, 0)

BS = 512


def _bigmm_body(a_ref, b_ref, o_ref):
    o_ref[...] = lax.dot_general(
        a_ref[...], b_ref[...], (((1,), (1,)), ((), ())),
        preferred_element_type=jnp.float32)


_bigmm = pl.pallas_call(
    _bigmm_body,
    grid=(pl.cdiv(N, BS), pl.cdiv(N, BS)),
    in_specs=[
        pl.BlockSpec((BS, H), lambda i, j: (i, 0)),
        pl.BlockSpec((BS, H), lambda i, j: (j, 0)),
    ],
    out_specs=pl.BlockSpec((BS, BS), lambda i, j: (i, j)),
    out_shape=jax.ShapeDtypeStruct((N, N), jnp.float32),
)


def kernel(x, edge_index, W_e1, b_e1, W_e2, b_e2, W_a1, b_a1, W_a2, b_a2,
           W_s, b_s):
    src = edge_index[0].astype(jnp.int32)
    dst = edge_index[1].astype(jnp.int32)
    pad = jnp.full((E_PAD - E,), N_PAD - 1, jnp.int32)
    srcp = jnp.concatenate([src, pad]).reshape(NS, CHT, K)
    dstp = jnp.concatenate([dst, pad]).reshape(NS, CHT, K)

    xp = jnp.zeros((N_PAD, D_IN), jnp.float32).at[:N].set(x)
    ones16 = jnp.ones((K, CD), jnp.float32)
    z16 = jnp.zeros((RPT, CD), jnp.float32)
    z64 = jnp.zeros((RPT, 64), jnp.float32)
    z128 = jnp.zeros((RPT, 128), jnp.float32)

    deg2 = _deg(dstp, ones16, z16)
    dinv = _finish_deg(deg2)

    u1 = _mm_scale_128_64(xp, W_e1, dinv)
    seg1 = _segsum64(u1, srcp, dstp, z64)
    h = _combine_64_relu(seg1, u1, dinv, b_e1.reshape(1, -1))

    u2 = _mm_scale_64_64(h, W_e2, dinv)
    seg2 = _segsum64(u2, srcp, dstp, z64)
    emb_p = _combine_64(seg2, u2, dinv, b_e2.reshape(1, -1))

    W3 = jnp.concatenate([W_a1, W_s], axis=1)
    b3 = jnp.concatenate([b_a1, b_s]).reshape(1, -1)
    u3 = _mm_scale_64_128(emb_p, W3, dinv)
    seg3 = _segsum128(u3, srcp, dstp, z128)
    t3 = _combine_128_relu64(seg3, u3, dinv, b3)
    a = t3[:, :H]
    hs = t3[:N, H:]

    u4 = _mm_scale_64_128(a, W_a2, dinv)
    seg4 = _segsum128(u4, srcp, dstp, z128)
    x_p = _combine_128(seg4, u4, dinv, b_a2.reshape(1, -1))

    s_ = _bigmm(hs, hs)
    return (x_p[:N], s_, emb_p[:N])


# speedup vs baseline: 1.2724x; 1.2724x over previous
import functools

import jax
import jax.numpy as jnp
from jax import lax
from jax.experimental import pallas as pl
from jax.experimental.pallas import tpu as pltpu
from jax.experimental.pallas import tpu_sc as plsc

N = 10000
E = 320000
D_IN = 128
H = 64
C = 128

NC = 2
NS = 16
K = 128
GS = 4
KG = K // GS
IB = 16
CH0 = 112
CH1 = 48
CHT = CH0 + CH1
NB0 = CH0 // IB
NB1 = CH1 // IB
E_PAD = K * CHT * NS
N_PAD = 10240
RPT = N_PAD // NS

_mesh = plsc.VectorSubcoreMesh(core_axis_name="c", subcore_axis_name="s")


def _make_segsum(CW, tc_tiling, ch0):
    params = pltpu.CompilerParams(use_tc_tiling_on_sc=tc_tiling)
    nb0, nb1 = ch0 // IB, (CHT - ch0) // IB

    @functools.partial(
        pl.kernel,
        out_type=jax.ShapeDtypeStruct((NC, N_PAD, CW), jnp.float32),
        mesh=_mesh,
        compiler_params=params,
        scratch_types=[
            pltpu.VMEM((IB, K), jnp.int32),
            pltpu.VMEM((IB, K), jnp.int32),
            pltpu.VMEM((K, CW), jnp.float32),
            pltpu.VMEM((K, CW), jnp.float32),
            pltpu.VMEM_SHARED((N_PAD, CW), jnp.float32),
            pltpu.SemaphoreType.DMA,
            pltpu.SemaphoreType.DMA,
            pltpu.SemaphoreType.DMA,
            pltpu.SemaphoreType.DMA,
        ],
    )
    def segsum(h_hbm, srcp_hbm, dstp_hbm, zeros_hbm, out_hbm,
               src_v, dst_v, rows0, rows1, acc, gsem0, gsem1, ssem0, ssem1):
        cid = lax.axis_index("c")
        sid = lax.axis_index("s")
        base = cid * ch0
        nbs = jnp.where(cid == 0, nb0, nb1)
        r0 = sid * RPT
        pltpu.sync_copy(zeros_hbm, acc.at[pl.ds(r0, RPT)])
        plsc.subcore_barrier()

        bufs = ((rows0, gsem0, ssem0), (rows1, gsem1, ssem1))

        def gather(src_v, j, rows, gsem):
            for g in range(GS):
                pltpu.async_copy(h_hbm.at[src_v.at[j, pl.ds(g * KG, KG)]],
                                 rows.at[pl.ds(g * KG, KG)], gsem)

        def gather_wait(src_v, j, rows, gsem):
            for g in range(GS):
                pltpu.make_async_copy(
                    h_hbm.at[src_v.at[j, pl.ds(g * KG, KG)]],
                    rows.at[pl.ds(g * KG, KG)], gsem).wait()

        def block(nb, carry):
            c0 = base + nb * IB
            pltpu.sync_copy(srcp_hbm.at[sid, pl.ds(c0, IB)], src_v)
            pltpu.sync_copy(dstp_hbm.at[sid, pl.ds(c0, IB)], dst_v)
            gather(src_v, 0, rows0, gsem0)
            for j in range(IB):
                rows, gsem, ssem = bufs[j % 2]
                prows, pgsem, pssem = bufs[(j + 1) % 2]
                gather_wait(src_v, j, rows, gsem)
                pltpu.async_copy(rows, acc.at[dst_v.at[j]], ssem, add=True)
                if j >= 1:
                    pltpu.make_async_copy(
                        prows, acc.at[dst_v.at[j - 1]], pssem).wait()
                if j + 1 < IB:
                    gather(src_v, j + 1, prows, pgsem)
            rows, gsem, ssem = bufs[(IB - 1) % 2]
            pltpu.make_async_copy(rows, acc.at[dst_v.at[IB - 1]], ssem).wait()
            return carry

        lax.fori_loop(0, nbs, block, 0)
        plsc.subcore_barrier()
        pltpu.sync_copy(acc.at[pl.ds(r0, RPT)],
                        out_hbm.at[cid, pl.ds(r0, RPT)])

    return segsum


_segsum64 = _make_segsum(64, False, 112)

CD = 16


@functools.partial(
    pl.kernel,
    out_type=jax.ShapeDtypeStruct((NC, N_PAD, CD), jnp.float32),
    mesh=_mesh,
    compiler_params=pltpu.CompilerParams(use_tc_tiling_on_sc=False),
    scratch_types=[
        pltpu.VMEM((IB, K), jnp.int32),
        pltpu.VMEM((K, CD), jnp.float32),
        pltpu.VMEM_SHARED((N_PAD, CD), jnp.float32),
    ],
)
def _deg(dstp_hbm, ones_hbm, zeros_hbm, out_hbm, dst_v, ones_v, acc):
    cid = lax.axis_index("c")
    sid = lax.axis_index("s")
    base = cid * CH0
    nbs = jnp.where(cid == 0, NB0, NB1)
    pltpu.sync_copy(ones_hbm, ones_v)
    r0 = sid * RPT
    pltpu.sync_copy(zeros_hbm, acc.at[pl.ds(r0, RPT)])
    plsc.subcore_barrier()

    def block(nb, carry):
        pltpu.sync_copy(dstp_hbm.at[sid, pl.ds(base + nb * IB, IB)], dst_v)
        for j in range(IB):
            pltpu.sync_copy(ones_v, acc.at[dst_v.at[j]], add=True)
        return carry

    lax.fori_loop(0, nbs, block, 0)
    plsc.subcore_barrier()
    pltpu.sync_copy(acc.at[pl.ds(r0, RPT)], out_hbm.at[cid, pl.ds(r0, RPT)])


BM = 2048


def _finish_deg_body(deg_ref, o_ref):
    d = deg_ref[...]
    o_ref[...] = lax.rsqrt(d[0, :, 0:1] + d[1, :, 0:1] + 1.0)


_finish_deg = pl.pallas_call(
    _finish_deg_body,
    grid=(N_PAD // BM,),
    in_specs=[pl.BlockSpec((NC, BM, CD), lambda i: (0, i, 0))],
    out_specs=pl.BlockSpec((BM, 1), lambda i: (i, 0)),
    out_shape=jax.ShapeDtypeStruct((N_PAD, 1), jnp.float32),
)


def _mm_scale_body(p_ref, w_ref, dv_ref, o_ref):
    o_ref[...] = dv_ref[...] * jnp.dot(
        p_ref[...], w_ref[...], preferred_element_type=jnp.float32)


def _make_mm_scale(cin, cout):
    return pl.pallas_call(
        _mm_scale_body,
        grid=(N_PAD // BM,),
        in_specs=[
            pl.BlockSpec((BM, cin), lambda i: (i, 0)),
            pl.BlockSpec((cin, cout), lambda i: (0, 0)),
            pl.BlockSpec((BM, 1), lambda i: (i, 0)),
        ],
        out_specs=pl.BlockSpec((BM, cout), lambda i: (i, 0)),
        out_shape=jax.ShapeDtypeStruct((N_PAD, cout), jnp.float32),
    )


_mm_scale_128_64 = _make_mm_scale(128, 64)
_mm_scale_64_64 = _make_mm_scale(64, 64)


def _combine_body(seg_ref, u_ref, dv_ref, b_ref, o_ref, *, crelu):
    t = dv_ref[...] * (seg_ref[0] + seg_ref[1] + u_ref[...]) + b_ref[...]
    if crelu == t.shape[1]:
        t = jnp.maximum(t, 0.0)
    elif crelu > 0:
        cols = lax.broadcasted_iota(jnp.int32, t.shape, 1)
        t = jnp.where(cols < crelu, jnp.maximum(t, 0.0), t)
    o_ref[...] = t


def _make_combine(cw, crelu):
    return pl.pallas_call(
        functools.partial(_combine_body, crelu=crelu),
        grid=(N_PAD // BM,),
        in_specs=[
            pl.BlockSpec((NC, BM, cw), lambda i: (0, i, 0)),
            pl.BlockSpec((BM, cw), lambda i: (i, 0)),
            pl.BlockSpec((BM, 1), lambda i: (i, 0)),
            pl.BlockSpec((1, cw), lambda i: (0, 0)),
        ],
        out_specs=pl.BlockSpec((BM, cw), lambda i: (i, 0)),
        out_shape=jax.ShapeDtypeStruct((N_PAD, cw), jnp.float32),
    )


_combine_64_relu = _make_combine(64, 64)
_combine_64 = _make_combine(64, 0)


def _scale_body(p_ref, dv_ref, o_ref):
    o_ref[...] = dv_ref[...] * p_ref[...]


_scale = pl.pallas_call(
    _scale_body,
    grid=(N_PAD // BM,),
    in_specs=[
        pl.BlockSpec((BM, H), lambda i: (i, 0)),
        pl.BlockSpec((BM, 1), lambda i: (i, 0)),
    ],
    out_specs=pl.BlockSpec((BM, H), lambda i: (i, 0)),
    out_shape=jax.ShapeDtypeStruct((N_PAD, H), jnp.float32),
)


def _mm_bias_body(p_ref, w_ref, b_ref, o_ref, *, crelu):
    t = jnp.dot(p_ref[...], w_ref[...],
                preferred_element_type=jnp.float32) + b_ref[...]
    if crelu > 0:
        cols = lax.broadcasted_iota(jnp.int32, t.shape, 1)
        t = jnp.where(cols < crelu, jnp.maximum(t, 0.0), t)
    o_ref[...] = t


def _make_mm_bias(crelu):
    return pl.pallas_call(
        functools.partial(_mm_bias_body, crelu=crelu),
        grid=(N_PAD // BM,),
        in_specs=[
            pl.BlockSpec((BM, H), lambda i: (i, 0)),
            pl.BlockSpec((H, D_IN), lambda i: (0, 0)),
            pl.BlockSpec((1, D_IN), lambda i: (0, 0)),
        ],
        out_specs=pl.BlockSpec((BM, D_IN), lambda i: (i, 0)),
        out_shape=jax.ShapeDtypeStruct((N_PAD, D_IN), jnp.float32),
    )


_mm_bias_relu64 = _make_mm_bias(64)
_mm_bias = _make_mm_bias(0)

BS = 512


def _bigmm_body(a_ref, b_ref, o_ref):
    o_ref[...] = lax.dot_general(
        a_ref[...], b_ref[...], (((1,), (1,)), ((), ())),
        preferred_element_type=jnp.float32)


_bigmm = pl.pallas_call(
    _bigmm_body,
    grid=(pl.cdiv(N, BS), pl.cdiv(N, BS)),
    in_specs=[
        pl.BlockSpec((BS, H), lambda i, j: (i, 0)),
        pl.BlockSpec((BS, H), lambda i, j: (j, 0)),
    ],
    out_specs=pl.BlockSpec((BS, BS), lambda i, j: (i, j)),
    out_shape=jax.ShapeDtypeStruct((N, N), jnp.float32),
)


def kernel(x, edge_index, W_e1, b_e1, W_e2, b_e2, W_a1, b_a1, W_a2, b_a2,
           W_s, b_s):
    src = edge_index[0].astype(jnp.int32)
    dst = edge_index[1].astype(jnp.int32)
    pad = jnp.full((E_PAD - E,), N_PAD - 1, jnp.int32)
    srcp = jnp.concatenate([src, pad]).reshape(NS, CHT, K)
    dstp = jnp.concatenate([dst, pad]).reshape(NS, CHT, K)

    xp = jnp.zeros((N_PAD, D_IN), jnp.float32).at[:N].set(x)
    ones16 = jnp.ones((K, CD), jnp.float32)
    z16 = jnp.zeros((RPT, CD), jnp.float32)
    z64 = jnp.zeros((RPT, 64), jnp.float32)

    deg2 = _deg(dstp, ones16, z16)
    dinv = _finish_deg(deg2)

    u1 = _mm_scale_128_64(xp, W_e1, dinv)
    seg1 = _segsum64(u1, srcp, dstp, z64)
    h = _combine_64_relu(seg1, u1, dinv, b_e1.reshape(1, -1))

    u2 = _mm_scale_64_64(h, W_e2, dinv)
    seg2 = _segsum64(u2, srcp, dstp, z64)
    emb_p = _combine_64(seg2, u2, dinv, b_e2.reshape(1, -1))

    zb = jnp.zeros((1, H), jnp.float32)
    W3 = jnp.concatenate([W_a1, W_s], axis=1)
    b3 = jnp.concatenate([b_a1, b_s]).reshape(1, -1)
    v3 = _scale(emb_p, dinv)
    seg3 = _segsum64(v3, srcp, dstp, z64)
    p3 = _combine_64(seg3, v3, dinv, zb)
    t3 = _mm_bias_relu64(p3, W3, b3)
    a = t3[:, :H]
    hs = t3[:N, H:]

    v4 = _scale(a, dinv)
    seg4 = _segsum64(v4, srcp, dstp, z64)
    p4 = _combine_64(seg4, v4, dinv, zb)
    x_p = _mm_bias(p4, W_a2, b_a2.reshape(1, -1))

    s_ = _bigmm(hs, hs)
    return (x_p[:N], s_, emb_p[:N])


# speedup vs baseline: 1.3505x; 1.0614x over previous
import functools

import jax
import jax.numpy as jnp
from jax import lax
from jax.experimental import pallas as pl
from jax.experimental.pallas import tpu as pltpu
from jax.experimental.pallas import tpu_sc as plsc

N = 10000
E = 320000
D_IN = 128
H = 64
C = 128

NC = 2
NS = 16
K = 128
GS = 4
KG = K // GS
IB = 16
CH0 = 112
CH1 = 48
CHT = CH0 + CH1
NB0 = CH0 // IB
NB1 = CH1 // IB
E_PAD = K * CHT * NS
N_PAD = 10240
RPT = N_PAD // NS

_mesh = plsc.VectorSubcoreMesh(core_axis_name="c", subcore_axis_name="s")


def _make_segsum(CW, tc_tiling, ch0):
    params = pltpu.CompilerParams(use_tc_tiling_on_sc=tc_tiling)
    nb0, nb1 = ch0 // IB, (CHT - ch0) // IB

    @functools.partial(
        pl.kernel,
        out_type=jax.ShapeDtypeStruct((NC, N_PAD, CW), jnp.float32),
        mesh=_mesh,
        compiler_params=params,
        scratch_types=[
            pltpu.VMEM((IB, K), jnp.int32),
            pltpu.VMEM((IB, K), jnp.int32),
            pltpu.VMEM((K, CW), jnp.float32),
            pltpu.VMEM((K, CW), jnp.float32),
            pltpu.VMEM_SHARED((N_PAD, CW), jnp.float32),
            pltpu.SemaphoreType.DMA,
            pltpu.SemaphoreType.DMA,
            pltpu.SemaphoreType.DMA,
            pltpu.SemaphoreType.DMA,
        ],
    )
    def segsum(h_hbm, srcp_hbm, dstp_hbm, zeros_hbm, out_hbm,
               src_v, dst_v, rows0, rows1, acc, gsem0, gsem1, ssem0, ssem1):
        cid = lax.axis_index("c")
        sid = lax.axis_index("s")
        base = cid * ch0
        nbs = jnp.where(cid == 0, nb0, nb1)
        r0 = sid * RPT
        pltpu.sync_copy(zeros_hbm, acc.at[pl.ds(r0, RPT)])
        plsc.subcore_barrier()

        bufs = ((rows0, gsem0, ssem0), (rows1, gsem1, ssem1))

        def gather(src_v, j, rows, gsem):
            for g in range(GS):
                pltpu.async_copy(h_hbm.at[src_v.at[j, pl.ds(g * KG, KG)]],
                                 rows.at[pl.ds(g * KG, KG)], gsem)

        def gather_wait(src_v, j, rows, gsem):
            for g in range(GS):
                pltpu.make_async_copy(
                    h_hbm.at[src_v.at[j, pl.ds(g * KG, KG)]],
                    rows.at[pl.ds(g * KG, KG)], gsem).wait()

        def block(nb, carry):
            c0 = base + nb * IB
            pltpu.sync_copy(srcp_hbm.at[sid, pl.ds(c0, IB)], src_v)
            pltpu.sync_copy(dstp_hbm.at[sid, pl.ds(c0, IB)], dst_v)
            gather(src_v, 0, rows0, gsem0)
            for j in range(IB):
                rows, gsem, ssem = bufs[j % 2]
                prows, pgsem, pssem = bufs[(j + 1) % 2]
                gather_wait(src_v, j, rows, gsem)
                pltpu.async_copy(rows, acc.at[dst_v.at[j]], ssem, add=True)
                if j >= 1:
                    pltpu.make_async_copy(
                        prows, acc.at[dst_v.at[j - 1]], pssem).wait()
                if j + 1 < IB:
                    gather(src_v, j + 1, prows, pgsem)
            rows, gsem, ssem = bufs[(IB - 1) % 2]
            pltpu.make_async_copy(rows, acc.at[dst_v.at[IB - 1]], ssem).wait()
            return carry

        lax.fori_loop(0, nbs, block, 0)
        plsc.subcore_barrier()
        pltpu.sync_copy(acc.at[pl.ds(r0, RPT)],
                        out_hbm.at[cid, pl.ds(r0, RPT)])

    return segsum


_segsum64 = _make_segsum(64, False, 112)

CD = 16


@functools.partial(
    pl.kernel,
    out_type=jax.ShapeDtypeStruct((NC, N_PAD, CD), jnp.float32),
    mesh=_mesh,
    compiler_params=pltpu.CompilerParams(use_tc_tiling_on_sc=False),
    scratch_types=[
        pltpu.VMEM((IB, K), jnp.int32),
        pltpu.VMEM((K, CD), jnp.float32),
        pltpu.VMEM_SHARED((N_PAD, CD), jnp.float32),
    ],
)
def _deg(dstp_hbm, ones_hbm, zeros_hbm, out_hbm, dst_v, ones_v, acc):
    cid = lax.axis_index("c")
    sid = lax.axis_index("s")
    base = cid * CH0
    nbs = jnp.where(cid == 0, NB0, NB1)
    pltpu.sync_copy(ones_hbm, ones_v)
    r0 = sid * RPT
    pltpu.sync_copy(zeros_hbm, acc.at[pl.ds(r0, RPT)])
    plsc.subcore_barrier()

    def block(nb, carry):
        pltpu.sync_copy(dstp_hbm.at[sid, pl.ds(base + nb * IB, IB)], dst_v)
        for j in range(IB):
            pltpu.sync_copy(ones_v, acc.at[dst_v.at[j]], add=True)
        return carry

    lax.fori_loop(0, nbs, block, 0)
    plsc.subcore_barrier()
    pltpu.sync_copy(acc.at[pl.ds(r0, RPT)], out_hbm.at[cid, pl.ds(r0, RPT)])


BM = 2048


def _finish_deg_body(deg_ref, o_ref):
    d = deg_ref[...]
    o_ref[...] = lax.rsqrt(d[0, :, 0:1] + d[1, :, 0:1] + 1.0)


_finish_deg = pl.pallas_call(
    _finish_deg_body,
    grid=(N_PAD // BM,),
    in_specs=[pl.BlockSpec((NC, BM, CD), lambda i: (0, i, 0))],
    out_specs=pl.BlockSpec((BM, 1), lambda i: (i, 0)),
    out_shape=jax.ShapeDtypeStruct((N_PAD, 1), jnp.float32),
)


def _mm_scale_body(p_ref, w_ref, dv_ref, o_ref):
    o_ref[...] = dv_ref[...] * jnp.dot(
        p_ref[...], w_ref[...], preferred_element_type=jnp.float32)


def _make_mm_scale(cin, cout):
    return pl.pallas_call(
        _mm_scale_body,
        grid=(N_PAD // BM,),
        in_specs=[
            pl.BlockSpec((BM, cin), lambda i: (i, 0)),
            pl.BlockSpec((cin, cout), lambda i: (0, 0)),
            pl.BlockSpec((BM, 1), lambda i: (i, 0)),
        ],
        out_specs=pl.BlockSpec((BM, cout), lambda i: (i, 0)),
        out_shape=jax.ShapeDtypeStruct((N_PAD, cout), jnp.float32),
    )


_mm_scale_128_64 = _make_mm_scale(128, 64)
_mm_scale_64_64 = _make_mm_scale(64, 64)


def _combine_body(seg_ref, u_ref, dv_ref, b_ref, o_ref, *, crelu):
    t = dv_ref[...] * (seg_ref[0] + seg_ref[1] + u_ref[...]) + b_ref[...]
    if crelu == t.shape[1]:
        t = jnp.maximum(t, 0.0)
    elif crelu > 0:
        cols = lax.broadcasted_iota(jnp.int32, t.shape, 1)
        t = jnp.where(cols < crelu, jnp.maximum(t, 0.0), t)
    o_ref[...] = t


def _make_combine(cw, crelu):
    return pl.pallas_call(
        functools.partial(_combine_body, crelu=crelu),
        grid=(N_PAD // BM,),
        in_specs=[
            pl.BlockSpec((NC, BM, cw), lambda i: (0, i, 0)),
            pl.BlockSpec((BM, cw), lambda i: (i, 0)),
            pl.BlockSpec((BM, 1), lambda i: (i, 0)),
            pl.BlockSpec((1, cw), lambda i: (0, 0)),
        ],
        out_specs=pl.BlockSpec((BM, cw), lambda i: (i, 0)),
        out_shape=jax.ShapeDtypeStruct((N_PAD, cw), jnp.float32),
    )


_combine_64_relu = _make_combine(64, 64)
_combine_64 = _make_combine(64, 0)


def _scale_body(p_ref, dv_ref, o_ref):
    o_ref[...] = dv_ref[...] * p_ref[...]


_scale = pl.pallas_call(
    _scale_body,
    grid=(N_PAD // BM,),
    in_specs=[
        pl.BlockSpec((BM, H), lambda i: (i, 0)),
        pl.BlockSpec((BM, 1), lambda i: (i, 0)),
    ],
    out_specs=pl.BlockSpec((BM, H), lambda i: (i, 0)),
    out_shape=jax.ShapeDtypeStruct((N_PAD, H), jnp.float32),
)


def _mm_bias_body(p_ref, w_ref, b_ref, o_ref, *, crelu):
    t = jnp.dot(p_ref[...], w_ref[...], precision=lax.Precision.HIGHEST,
                preferred_element_type=jnp.float32) + b_ref[...]
    if crelu > 0:
        cols = lax.broadcasted_iota(jnp.int32, t.shape, 1)
        t = jnp.where(cols < crelu, jnp.maximum(t, 0.0), t)
    o_ref[...] = t


def _make_mm_bias(crelu):
    return pl.pallas_call(
        functools.partial(_mm_bias_body, crelu=crelu),
        grid=(N_PAD // BM,),
        in_specs=[
            pl.BlockSpec((BM, H), lambda i: (i, 0)),
            pl.BlockSpec((H, D_IN), lambda i: (0, 0)),
            pl.BlockSpec((1, D_IN), lambda i: (0, 0)),
        ],
        out_specs=pl.BlockSpec((BM, D_IN), lambda i: (i, 0)),
        out_shape=jax.ShapeDtypeStruct((N_PAD, D_IN), jnp.float32),
    )


_mm_bias_relu64 = _make_mm_bias(64)
_mm_bias = _make_mm_bias(0)

BS = 512


def _bigmm_body(a_ref, b_ref, o_ref):
    o_ref[...] = lax.dot_general(
        a_ref[...], b_ref[...], (((1,), (1,)), ((), ())),
        preferred_element_type=jnp.float32)


_bigmm = pl.pallas_call(
    _bigmm_body,
    grid=(pl.cdiv(N, BS), pl.cdiv(N, BS)),
    in_specs=[
        pl.BlockSpec((BS, H), lambda i, j: (i, 0)),
        pl.BlockSpec((BS, H), lambda i, j: (j, 0)),
    ],
    out_specs=pl.BlockSpec((BS, BS), lambda i, j: (i, j)),
    out_shape=jax.ShapeDtypeStruct((N, N), jnp.float32),
)


def kernel(x, edge_index, W_e1, b_e1, W_e2, b_e2, W_a1, b_a1, W_a2, b_a2,
           W_s, b_s):
    src = edge_index[0].astype(jnp.int32)
    dst = edge_index[1].astype(jnp.int32)
    pad = jnp.full((E_PAD - E,), N_PAD - 1, jnp.int32)
    srcp = jnp.concatenate([src, pad]).reshape(NS, CHT, K)
    dstp = jnp.concatenate([dst, pad]).reshape(NS, CHT, K)

    xp = jnp.zeros((N_PAD, D_IN), jnp.float32).at[:N].set(x)
    ones16 = jnp.ones((K, CD), jnp.float32)
    z16 = jnp.zeros((RPT, CD), jnp.float32)
    z64 = jnp.zeros((RPT, 64), jnp.float32)

    deg2 = _deg(dstp, ones16, z16)
    dinv = _finish_deg(deg2)

    u1 = _mm_scale_128_64(xp, W_e1, dinv)
    seg1 = _segsum64(u1, srcp, dstp, z64)
    h = _combine_64_relu(seg1, u1, dinv, b_e1.reshape(1, -1))

    u2 = _mm_scale_64_64(h, W_e2, dinv)
    seg2 = _segsum64(u2, srcp, dstp, z64)
    emb_p = _combine_64(seg2, u2, dinv, b_e2.reshape(1, -1))

    zb = jnp.zeros((1, H), jnp.float32)
    W3 = jnp.concatenate([W_a1, W_s], axis=1)
    b3 = jnp.concatenate([b_a1, b_s]).reshape(1, -1)
    v3 = _scale(emb_p, dinv)
    seg3 = _segsum64(v3, srcp, dstp, z64)
    p3 = _combine_64(seg3, v3, dinv, zb)
    t3 = _mm_bias_relu64(p3, W3, b3)
    a = t3[:, :H]
    hs = t3[:N, H:]

    v4 = _scale(a, dinv)
    seg4 = _segsum64(v4, srcp, dstp, z64)
    p4 = _combine_64(seg4, v4, dinv, zb)
    x_p = _mm_bias(p4, W_a2, b_a2.reshape(1, -1))

    s_ = _bigmm(hs, hs)
    return (x_p[:N], s_, emb_p[:N])
